# baseline (speedup 1.0000x reference)
import functools

import jax
import jax.numpy as jnp
from jax import lax
from jax.experimental import pallas as pl
from jax.experimental.pallas import tpu as pltpu
from jax.experimental.pallas import tpu_sc as plsc

_INFO = plsc.get_sparse_core_info()
_NC = _INFO.num_cores
_NS = _INFO.num_subcores
_NW = _NC * _NS
_L = _INFO.num_lanes

_BATCH = 16384
_BPW = _BATCH // _NW
_CHUNK = 128
_NCHUNK = _BPW // _CHUNK
_NCOL = 10

_N = 1000000
_TILES = 7812
_TPW = _TILES // _NW
_ACHUNK = 4
_CW = (_TPW // _ACHUNK) * 128
_R = 7813 * 128
_TAIL = _TILES * 128


def _detile_body(a4, a3, b3, t4, t3a, t3b, o4, o3a, o3b,
                 buf4, buf3, r0, r1, r2, r3, tb4, tb3, sem):
    wid = lax.axis_index("s") * _NC + lax.axis_index("c")
    base = wid * _TPW * 128
    rows = (r0, r1, r2, r3)

    def extract(buf, d):
        def step(i, _):
            for c in range(d):
                rows[c][pl.ds(i * _L, _L)] = buf[c, pl.ds(i * _L, _L)]
            return 0
        lax.fori_loop(0, _CW // _L, step, 0)

    writes = []
    for k in range(_ACHUNK):
        off = base + k * _CW
        for w in writes:
            w.wait()
        writes = []
        pltpu.sync_copy(a4.at[:, pl.ds(off, _CW)], buf4)
        extract(buf4, 4)
        for c in range(4):
            writes.append(pltpu.async_copy(
                rows[c], o4.at[pl.ds(c * _R + off, _CW)], sem))
        for w in writes:
            w.wait()
        writes = []
        for src, out, nrow in ((a3, o3a, 0), (b3, o3b, 1)):
            pltpu.sync_copy(src.at[:, pl.ds(off, _CW)], buf3)
            for w in writes:
                w.wait()
            writes = []
            extract(buf3, 3)
            for c in range(3):
                writes.append(pltpu.async_copy(
                    rows[c], out.at[pl.ds(c * _R + off, _CW)], sem))
        for w in writes:
            w.wait()
        writes = []

    @pl.when(wid < 4)
    def _():
        toff = (_TILES - 4) * 128 + wid * 128
        pltpu.sync_copy(a4.at[:, pl.ds(toff, 128)],
                        buf4.at[:, pl.ds(0, 128)])

        def tstep4(i, _):
            for c in range(4):
                rows[c][pl.ds(i * _L, _L)] = buf4[c, pl.ds(i * _L, _L)]
            return 0
        lax.fori_loop(0, 128 // _L, tstep4, 0)
        for c in range(4):
            pltpu.sync_copy(rows[c].at[pl.ds(0, 128)],
                            o4.at[pl.ds(c * _R + toff, 128)])
        for src, out in ((a3, o3a), (b3, o3b)):
            pltpu.sync_copy(src.at[:, pl.ds(toff, 128)],
                            buf3.at[:, pl.ds(0, 128)])

            def tstep3(i, _):
                for c in range(3):
                    rows[c][pl.ds(i * _L, _L)] = buf3[c, pl.ds(i * _L, _L)]
                return 0
            lax.fori_loop(0, 128 // _L, tstep3, 0)
            for c in range(3):
                pltpu.sync_copy(rows[c].at[pl.ds(0, 128)],
                                out.at[pl.ds(c * _R + toff, 128)])

    @pl.when(wid == 4)
    def _():
        pltpu.sync_copy(t4, tb4)
        for c in range(4):
            pltpu.sync_copy(tb4.at[pl.ds(c * 128, 128)],
                            o4.at[pl.ds(c * _R + _TAIL, 128)])

    @pl.when(wid == 5)
    def _():
        pltpu.sync_copy(t3a, tb3)
        for c in range(3):
            pltpu.sync_copy(tb3.at[pl.ds(c * 128, 128)],
                            o3a.at[pl.ds(c * _R + _TAIL, 128)])

    @pl.when(wid == 6)
    def _():
        pltpu.sync_copy(t3b, tb3)
        for c in range(3):
            pltpu.sync_copy(tb3.at[pl.ds(c * 128, 128)],
                            o3b.at[pl.ds(c * _R + _TAIL, 128)])


def _gather_body(idx_hbm, f4, f3a, f3b, out_hbm, idx_v, cols_v, stage_v, sem):
    wid = lax.axis_index("s") * _NC + lax.axis_index("c")
    base = wid * _BPW

    pltpu.sync_copy(idx_hbm.at[pl.ds(wid * _NCHUNK, _NCHUNK)], idx_v)

    srcs = ([f4.at[pl.ds(c * _R, _R)] for c in range(4)]
            + [f3a.at[pl.ds(c * _R, _R)] for c in range(3)]
            + [f3b.at[pl.ds(c * _R, _R)] for c in range(3)])
    copies = []
    for c in range(_NCOL):
        for k in range(_NCHUNK):
            copies.append(pltpu.async_copy(
                srcs[c].at[idx_v.at[k]],
                cols_v.at[c].at[pl.ds(k * _CHUNK, _CHUNK)],
                sem))
    for cp in copies:
        cp.wait()

    lanes = jnp.arange(_L, dtype=jnp.int32)

    def interleave(g, _):
        rows = g * _L + lanes
        for c in range(_NCOL):
            vals = cols_v.at[c][pl.ds(g * _L, _L)]
            ocol = jnp.full((_L,), c, dtype=jnp.int32)
            plsc.store_scatter(stage_v, [rows, ocol], vals)
        return 0

    lax.fori_loop(0, _BPW // _L, interleave, 0)

    pltpu.sync_copy(stage_v, out_hbm.at[pl.ds(base, _BPW)])


@jax.jit
def _run(camera_idx, intrinsic_deltas, rotation_deltas, translation_deltas):
    mesh = plsc.VectorSubcoreMesh(core_axis_name="c", subcore_axis_name="s")

    tail4 = jnp.ravel(
        jnp.pad(intrinsic_deltas[_TAIL:], ((0, 64), (0, 0))).T)
    tail3a = jnp.ravel(
        jnp.pad(rotation_deltas[_TAIL:], ((0, 64), (0, 0))).T)
    tail3b = jnp.ravel(
        jnp.pad(translation_deltas[_TAIL:], ((0, 64), (0, 0))).T)

    detile = functools.partial(
        pl.kernel,
        out_type=(jax.ShapeDtypeStruct((4 * _R,), jnp.float32),
                  jax.ShapeDtypeStruct((3 * _R,), jnp.float32),
                  jax.ShapeDtypeStruct((3 * _R,), jnp.float32)),
        mesh=mesh,
        scratch_types=[
            pltpu.VMEM((4, _CW), jnp.float32),
            pltpu.VMEM((3, _CW), jnp.float32),
            pltpu.VMEM((_CW,), jnp.float32),
            pltpu.VMEM((_CW,), jnp.float32),
            pltpu.VMEM((_CW,), jnp.float32),
            pltpu.VMEM((_CW,), jnp.float32),
            pltpu.VMEM((4 * 128,), jnp.float32),
            pltpu.VMEM((3 * 128,), jnp.float32),
            pltpu.SemaphoreType.DMA,
        ],
        compiler_params=pltpu.CompilerParams(
            use_tc_tiling_on_sc=True, needs_layout_passes=False),
    )(_detile_body)
    f4, f3a, f3b = detile(intrinsic_deltas.T, rotation_deltas.T,
                          translation_deltas.T, tail4, tail3a, tail3b)

    gather = functools.partial(
        pl.kernel,
        out_type=jax.ShapeDtypeStruct((_BATCH, _NCOL), jnp.float32),
        mesh=mesh,
        scratch_types=[
            pltpu.VMEM((_NCHUNK, _CHUNK), jnp.int32),
            pltpu.VMEM((_NCOL, _BPW), jnp.float32),
            pltpu.VMEM((_BPW, _NCOL), jnp.float32),
            pltpu.SemaphoreType.DMA,
        ],
        compiler_params=pltpu.CompilerParams(
            use_tc_tiling_on_sc=False, needs_layout_passes=False),
    )(_gather_body)
    idx2 = camera_idx.reshape(_NW * _NCHUNK, _CHUNK)
    return gather(idx2, f4, f3a, f3b)


def kernel(camera_idx, intrinsic_deltas, rotation_deltas, translation_deltas):
    return _run(camera_idx.astype(jnp.int32), intrinsic_deltas,
                rotation_deltas, translation_deltas)


# speedup vs baseline: 1.3738x; 1.3738x over previous
import functools

import jax
import jax.numpy as jnp
from jax import lax
from jax.experimental import pallas as pl
from jax.experimental.pallas import tpu as pltpu
from jax.experimental.pallas import tpu_sc as plsc

_INFO = plsc.get_sparse_core_info()
_NC = _INFO.num_cores
_NS = _INFO.num_subcores
_NW = _NC * _NS
_L = _INFO.num_lanes

_BATCH = 16384
_BPW = _BATCH // _NW
_CHUNK = 128
_NCHUNK = _BPW // _CHUNK
_NCOL = 10

_N = 1000000
_TILES = 7812
_TPW = _TILES // _NW
_ACHUNK = 4
_CW = (_TPW // _ACHUNK) * 128
_R = 7813 * 128
_TAIL = _TILES * 128


def _detile_body(a3, b3, t3a, t3b, o3a, o3b, buf3, r0, r1, r2, tb3, sem):
    wid = lax.axis_index("s") * _NC + lax.axis_index("c")
    base = wid * _TPW * 128
    rows = (r0, r1, r2)

    def extract(buf):
        def step(i, _):
            for c in range(3):
                rows[c][pl.ds(i * _L, _L)] = buf[c, pl.ds(i * _L, _L)]
            return 0
        lax.fori_loop(0, _CW // _L, step, 0)

    writes = []
    for k in range(_ACHUNK):
        off = base + k * _CW
        for src, out in ((a3, o3a), (b3, o3b)):
            pltpu.sync_copy(src.at[:, pl.ds(off, _CW)], buf3)
            for w in writes:
                w.wait()
            writes = []
            extract(buf3)
            for c in range(3):
                writes.append(pltpu.async_copy(
                    rows[c], out.at[pl.ds(c * _R + off, _CW)], sem))
    for w in writes:
        w.wait()

    @pl.when(wid < 4)
    def _():
        toff = (_TILES - 4) * 128 + wid * 128
        for src, out in ((a3, o3a), (b3, o3b)):
            pltpu.sync_copy(src.at[:, pl.ds(toff, 128)],
                            buf3.at[:, pl.ds(0, 128)])

            def tstep(i, _):
                for c in range(3):
                    rows[c][pl.ds(i * _L, _L)] = buf3[c, pl.ds(i * _L, _L)]
                return 0
            lax.fori_loop(0, 128 // _L, tstep, 0)
            for c in range(3):
                pltpu.sync_copy(rows[c].at[pl.ds(0, 128)],
                                out.at[pl.ds(c * _R + toff, 128)])

    @pl.when(wid == 4)
    def _():
        pltpu.sync_copy(t3a, tb3)
        for c in range(3):
            pltpu.sync_copy(tb3.at[pl.ds(c * 128, 128)],
                            o3a.at[pl.ds(c * _R + _TAIL, 128)])

    @pl.when(wid == 5)
    def _():
        pltpu.sync_copy(t3b, tb3)
        for c in range(3):
            pltpu.sync_copy(tb3.at[pl.ds(c * 128, 128)],
                            o3b.at[pl.ds(c * _R + _TAIL, 128)])


def _gather_body(idx_hbm, i4, f3a, f3b, out_hbm, idx_v, cols_v, stage_v, sem):
    wid = lax.axis_index("s") * _NC + lax.axis_index("c")
    base = wid * _BPW

    pltpu.sync_copy(idx_hbm.at[pl.ds(wid * _NCHUNK, _NCHUNK)], idx_v)

    srcs = ([i4.at[c] for c in range(4)]
            + [f3a.at[pl.ds(c * _R, _R)] for c in range(3)]
            + [f3b.at[pl.ds(c * _R, _R)] for c in range(3)])
    copies = []
    for c in range(_NCOL):
        for k in range(_NCHUNK):
            copies.append(pltpu.async_copy(
                srcs[c].at[idx_v.at[k]],
                cols_v.at[c].at[pl.ds(k * _CHUNK, _CHUNK)],
                sem))
    for cp in copies:
        cp.wait()

    lanes = jnp.arange(_L, dtype=jnp.int32)

    def interleave(g, _):
        rows = g * _L + lanes
        for c in range(_NCOL):
            vals = cols_v.at[c][pl.ds(g * _L, _L)]
            ocol = jnp.full((_L,), c, dtype=jnp.int32)
            plsc.store_scatter(stage_v, [rows, ocol], vals)
        return 0

    lax.fori_loop(0, _BPW // _L, interleave, 0)

    pltpu.sync_copy(stage_v, out_hbm.at[pl.ds(base, _BPW)])


@jax.jit
def _run(camera_idx, intrinsic_deltas, rotation_deltas, translation_deltas):
    mesh = plsc.VectorSubcoreMesh(core_axis_name="c", subcore_axis_name="s")

    tail3a = jnp.ravel(
        jnp.pad(rotation_deltas[_TAIL:], ((0, 64), (0, 0))).T)
    tail3b = jnp.ravel(
        jnp.pad(translation_deltas[_TAIL:], ((0, 64), (0, 0))).T)

    detile = functools.partial(
        pl.kernel,
        out_type=(jax.ShapeDtypeStruct((3 * _R,), jnp.float32),
                  jax.ShapeDtypeStruct((3 * _R,), jnp.float32)),
        mesh=mesh,
        scratch_types=[
            pltpu.VMEM((3, _CW), jnp.float32),
            pltpu.VMEM((_CW,), jnp.float32),
            pltpu.VMEM((_CW,), jnp.float32),
            pltpu.VMEM((_CW,), jnp.float32),
            pltpu.VMEM((3 * 128,), jnp.float32),
            pltpu.SemaphoreType.DMA,
        ],
        compiler_params=pltpu.CompilerParams(
            use_tc_tiling_on_sc=True, needs_layout_passes=False),
    )(_detile_body)
    f3a, f3b = detile(rotation_deltas.T, translation_deltas.T,
                      tail3a, tail3b)

    gather = functools.partial(
        pl.kernel,
        out_type=jax.ShapeDtypeStruct((_BATCH, _NCOL), jnp.float32),
        mesh=mesh,
        scratch_types=[
            pltpu.VMEM((_NCHUNK, _CHUNK), jnp.int32),
            pltpu.VMEM((_NCOL, _BPW), jnp.float32),
            pltpu.VMEM((_BPW, _NCOL), jnp.float32),
            pltpu.SemaphoreType.DMA,
        ],
        compiler_params=pltpu.CompilerParams(
            use_tc_tiling_on_sc=False, needs_layout_passes=False),
    )(_gather_body)
    idx2 = camera_idx.reshape(_NW * _NCHUNK, _CHUNK)
    return gather(idx2, intrinsic_deltas.T, f3a, f3b)


def kernel(camera_idx, intrinsic_deltas, rotation_deltas, translation_deltas):
    return _run(camera_idx.astype(jnp.int32), intrinsic_deltas,
                rotation_deltas, translation_deltas)


# speedup vs baseline: 1.3802x; 1.0046x over previous
import functools

import jax
import jax.numpy as jnp
from jax import lax
from jax.experimental import pallas as pl
from jax.experimental.pallas import tpu as pltpu
from jax.experimental.pallas import tpu_sc as plsc

_INFO = plsc.get_sparse_core_info()
_NC = _INFO.num_cores
_NS = _INFO.num_subcores
_NW = _NC * _NS
_L = _INFO.num_lanes

_BATCH = 16384
_BPW = _BATCH // _NW
_CHUNK = 128
_NCHUNK = _BPW // _CHUNK
_NCOL = 10

_N = 1000000
_TILES = 7812
_TPW = _TILES // _NW
_ACHUNK = 4
_CW = (_TPW // _ACHUNK) * 128
_R = 7813 * 128
_TAIL = _TILES * 128


def _detile_body(a3, b3, t3a, t3b, o3a, o3b, buf3, r0, r1, r2, tb3, sem):
    wid = lax.axis_index("s") * _NC + lax.axis_index("c")
    base = wid * _TPW * 128
    rows = (r0, r1, r2)

    def extract(buf):
        def step(i, _):
            for u in range(8):
                o = (i * 8 + u) * _L
                for c in range(3):
                    rows[c][pl.ds(o, _L)] = buf[c, pl.ds(o, _L)]
            return 0
        lax.fori_loop(0, _CW // (_L * 8), step, 0)

    writes = []
    for k in range(_ACHUNK):
        off = base + k * _CW
        for src, out in ((a3, o3a), (b3, o3b)):
            pltpu.sync_copy(src.at[:, pl.ds(off, _CW)], buf3)
            for w in writes:
                w.wait()
            writes = []
            extract(buf3)
            for c in range(3):
                writes.append(pltpu.async_copy(
                    rows[c], out.at[pl.ds(c * _R + off, _CW)], sem))
    for w in writes:
        w.wait()

    @pl.when(wid < 4)
    def _():
        toff = (_TILES - 4) * 128 + wid * 128
        for src, out in ((a3, o3a), (b3, o3b)):
            pltpu.sync_copy(src.at[:, pl.ds(toff, 128)],
                            buf3.at[:, pl.ds(0, 128)])

            def tstep(i, _):
                for c in range(3):
                    rows[c][pl.ds(i * _L, _L)] = buf3[c, pl.ds(i * _L, _L)]
                return 0
            lax.fori_loop(0, 128 // _L, tstep, 0)
            for c in range(3):
                pltpu.sync_copy(rows[c].at[pl.ds(0, 128)],
                                out.at[pl.ds(c * _R + toff, 128)])

    @pl.when(wid == 4)
    def _():
        pltpu.sync_copy(t3a, tb3)
        for c in range(3):
            pltpu.sync_copy(tb3.at[pl.ds(c * 128, 128)],
                            o3a.at[pl.ds(c * _R + _TAIL, 128)])

    @pl.when(wid == 5)
    def _():
        pltpu.sync_copy(t3b, tb3)
        for c in range(3):
            pltpu.sync_copy(tb3.at[pl.ds(c * 128, 128)],
                            o3b.at[pl.ds(c * _R + _TAIL, 128)])


def _gather_body(idx_hbm, i4, f3a, f3b, out_hbm, idx_v, cols_v, stage_v, sem):
    wid = lax.axis_index("s") * _NC + lax.axis_index("c")
    base = wid * _BPW

    pltpu.sync_copy(idx_hbm.at[pl.ds(wid * _NCHUNK, _NCHUNK)], idx_v)

    srcs = ([i4.at[c] for c in range(4)]
            + [f3a.at[pl.ds(c * _R, _R)] for c in range(3)]
            + [f3b.at[pl.ds(c * _R, _R)] for c in range(3)])
    copies = []
    for c in range(_NCOL):
        for k in range(_NCHUNK):
            copies.append(pltpu.async_copy(
                srcs[c].at[idx_v.at[k]],
                cols_v.at[c].at[pl.ds(k * _CHUNK, _CHUNK)],
                sem))
    for cp in copies:
        cp.wait()

    lanes = jnp.arange(_L, dtype=jnp.int32)

    def interleave(g, _):
        rows = g * _L + lanes
        for c in range(_NCOL):
            vals = cols_v.at[c][pl.ds(g * _L, _L)]
            ocol = jnp.full((_L,), c, dtype=jnp.int32)
            plsc.store_scatter(stage_v, [rows, ocol], vals)
        return 0

    lax.fori_loop(0, _BPW // _L, interleave, 0)

    pltpu.sync_copy(stage_v, out_hbm.at[pl.ds(base, _BPW)])


@jax.jit
def _run(camera_idx, intrinsic_deltas, rotation_deltas, translation_deltas):
    mesh = plsc.VectorSubcoreMesh(core_axis_name="c", subcore_axis_name="s")

    tail3a = jnp.ravel(
        jnp.pad(rotation_deltas[_TAIL:], ((0, 64), (0, 0))).T)
    tail3b = jnp.ravel(
        jnp.pad(translation_deltas[_TAIL:], ((0, 64), (0, 0))).T)

    detile = functools.partial(
        pl.kernel,
        out_type=(jax.ShapeDtypeStruct((3 * _R,), jnp.float32),
                  jax.ShapeDtypeStruct((3 * _R,), jnp.float32)),
        mesh=mesh,
        scratch_types=[
            pltpu.VMEM((3, _CW), jnp.float32),
            pltpu.VMEM((_CW,), jnp.float32),
            pltpu.VMEM((_CW,), jnp.float32),
            pltpu.VMEM((_CW,), jnp.float32),
            pltpu.VMEM((3 * 128,), jnp.float32),
            pltpu.SemaphoreType.DMA,
        ],
        compiler_params=pltpu.CompilerParams(
            use_tc_tiling_on_sc=True, needs_layout_passes=False),
    )(_detile_body)
    f3a, f3b = detile(rotation_deltas.T, translation_deltas.T,
                      tail3a, tail3b)

    gather = functools.partial(
        pl.kernel,
        out_type=jax.ShapeDtypeStruct((_BATCH, _NCOL), jnp.float32),
        mesh=mesh,
        scratch_types=[
            pltpu.VMEM((_NCHUNK, _CHUNK), jnp.int32),
            pltpu.VMEM((_NCOL, _BPW), jnp.float32),
            pltpu.VMEM((_BPW, _NCOL), jnp.float32),
            pltpu.SemaphoreType.DMA,
        ],
        compiler_params=pltpu.CompilerParams(
            use_tc_tiling_on_sc=False, needs_layout_passes=False),
    )(_gather_body)
    idx2 = camera_idx.reshape(_NW * _NCHUNK, _CHUNK)
    return gather(idx2, intrinsic_deltas.T, f3a, f3b)


def kernel(camera_idx, intrinsic_deltas, rotation_deltas, translation_deltas):
    return _run(camera_idx.astype(jnp.int32), intrinsic_deltas,
                rotation_deltas, translation_deltas)


# speedup vs baseline: 1.4931x; 1.0818x over previous
import functools

import jax
import jax.numpy as jnp
from jax import lax
from jax.experimental import pallas as pl
from jax.experimental.pallas import tpu as pltpu
from jax.experimental.pallas import tpu_sc as plsc

_INFO = plsc.get_sparse_core_info()
_NC = _INFO.num_cores
_NS = _INFO.num_subcores
_NW = _NC * _NS
_L = _INFO.num_lanes

_BATCH = 16384
_BPW = _BATCH // _NW
_CHUNK = 128
_NCHUNK = _BPW // _CHUNK
_NCOL = 10

_N = 1000000
_TILES = 7812
_TPW = _TILES // _NW
_ACHUNK = 4
_CW = (_TPW // _ACHUNK) * 128
_R = 7813 * 128
_TAIL = _TILES * 128


def _detile_body(a3, b3, t3a, t3b, o3a, o3b, buf3, r0, r1, r2, tb3, sem):
    wid = lax.axis_index("s") * _NC + lax.axis_index("c")
    base = wid * _TPW * 128
    rows = (r0, r1, r2)

    def extract(buf):
        def step(i, _):
            for u in range(8):
                o = (i * 8 + u) * _L
                for c in range(3):
                    rows[c][pl.ds(o, _L)] = buf[c, pl.ds(o, _L)]
            return 0
        lax.fori_loop(0, _CW // (_L * 8), step, 0)

    writes = []
    for k in range(_ACHUNK):
        off = base + k * _CW
        for src, out in ((a3, o3a), (b3, o3b)):
            pltpu.sync_copy(src.at[:, pl.ds(off, _CW)], buf3)
            for w in writes:
                w.wait()
            writes = []
            extract(buf3)
            for c in range(3):
                writes.append(pltpu.async_copy(
                    rows[c], out.at[pl.ds(c * _R + off, _CW)], sem))
    for w in writes:
        w.wait()

    @pl.when(wid < 4)
    def _():
        toff = (_TILES - 4) * 128 + wid * 128
        for src, out in ((a3, o3a), (b3, o3b)):
            pltpu.sync_copy(src.at[:, pl.ds(toff, 128)],
                            buf3.at[:, pl.ds(0, 128)])

            def tstep(i, _):
                for c in range(3):
                    rows[c][pl.ds(i * _L, _L)] = buf3[c, pl.ds(i * _L, _L)]
                return 0
            lax.fori_loop(0, 128 // _L, tstep, 0)
            for c in range(3):
                pltpu.sync_copy(rows[c].at[pl.ds(0, 128)],
                                out.at[pl.ds(c * _R + toff, 128)])

    @pl.when(wid == 4)
    def _():
        pltpu.sync_copy(t3a, tb3)
        for c in range(3):
            pltpu.sync_copy(tb3.at[pl.ds(c * 128, 128)],
                            o3a.at[pl.ds(c * _R + _TAIL, 128)])

    @pl.when(wid == 5)
    def _():
        pltpu.sync_copy(t3b, tb3)
        for c in range(3):
            pltpu.sync_copy(tb3.at[pl.ds(c * 128, 128)],
                            o3b.at[pl.ds(c * _R + _TAIL, 128)])


def _gather_body(idx_hbm, i4, f3a, f3b, *refs):
    outs = refs[:_NCOL]
    idx_v, cols_v, sem = refs[_NCOL:]
    wid = lax.axis_index("s") * _NC + lax.axis_index("c")
    base = wid * _BPW

    pltpu.sync_copy(idx_hbm.at[pl.ds(wid * _NCHUNK, _NCHUNK)], idx_v)

    srcs = ([i4.at[c] for c in range(4)]
            + [f3a.at[pl.ds(c * _R, _R)] for c in range(3)]
            + [f3b.at[pl.ds(c * _R, _R)] for c in range(3)])
    copies = []
    for c in range(_NCOL):
        for k in range(_NCHUNK):
            copies.append(pltpu.async_copy(
                srcs[c].at[idx_v.at[k]],
                cols_v.at[c].at[pl.ds(k * _CHUNK, _CHUNK)],
                sem))
    for cp in copies:
        cp.wait()

    writes = [pltpu.async_copy(cols_v.at[c], outs[c].at[pl.ds(base, _BPW)],
                               sem)
              for c in range(_NCOL)]
    for w in writes:
        w.wait()


@jax.jit
def _run(camera_idx, intrinsic_deltas, rotation_deltas, translation_deltas):
    mesh = plsc.VectorSubcoreMesh(core_axis_name="c", subcore_axis_name="s")

    tail3a = jnp.ravel(
        jnp.pad(rotation_deltas[_TAIL:], ((0, 64), (0, 0))).T)
    tail3b = jnp.ravel(
        jnp.pad(translation_deltas[_TAIL:], ((0, 64), (0, 0))).T)

    detile = functools.partial(
        pl.kernel,
        out_type=(jax.ShapeDtypeStruct((3 * _R,), jnp.float32),
                  jax.ShapeDtypeStruct((3 * _R,), jnp.float32)),
        mesh=mesh,
        scratch_types=[
            pltpu.VMEM((3, _CW), jnp.float32),
            pltpu.VMEM((_CW,), jnp.float32),
            pltpu.VMEM((_CW,), jnp.float32),
            pltpu.VMEM((_CW,), jnp.float32),
            pltpu.VMEM((3 * 128,), jnp.float32),
            pltpu.SemaphoreType.DMA,
        ],
        compiler_params=pltpu.CompilerParams(
            use_tc_tiling_on_sc=True, needs_layout_passes=False),
    )(_detile_body)
    f3a, f3b = detile(rotation_deltas.T, translation_deltas.T,
                      tail3a, tail3b)

    gather = functools.partial(
        pl.kernel,
        out_type=tuple(jax.ShapeDtypeStruct((_BATCH,), jnp.float32)
                       for _ in range(_NCOL)),
        mesh=mesh,
        scratch_types=[
            pltpu.VMEM((_NCHUNK, _CHUNK), jnp.int32),
            pltpu.VMEM((_NCOL, _BPW), jnp.float32),
            pltpu.SemaphoreType.DMA,
        ],
        compiler_params=pltpu.CompilerParams(
            use_tc_tiling_on_sc=False, needs_layout_passes=False),
    )(_gather_body)
    idx2 = camera_idx.reshape(_NW * _NCHUNK, _CHUNK)
    cols = gather(idx2, intrinsic_deltas.T, f3a, f3b)
    return jnp.stack(cols, axis=1)


def kernel(camera_idx, intrinsic_deltas, rotation_deltas, translation_deltas):
    return _run(camera_idx.astype(jnp.int32), intrinsic_deltas,
                rotation_deltas, translation_deltas)


# speedup vs baseline: 1.7977x; 1.2040x over previous
import functools

import jax
import jax.numpy as jnp
from jax import lax
from jax.experimental import pallas as pl
from jax.experimental.pallas import tpu as pltpu
from jax.experimental.pallas import tpu_sc as plsc

_INFO = plsc.get_sparse_core_info()
_NC = _INFO.num_cores
_NS = _INFO.num_subcores
_NW = _NC * _NS
_L = _INFO.num_lanes

_BATCH = 16384
_BPW = _BATCH // _NW
_CHUNK = 128
_NCHUNK = _BPW // _CHUNK
_NCOL = 10

_N = 1000000
_TILES = 7812
_TPW = _TILES // _NW
_ACHUNK = 4
_CW = (_TPW // _ACHUNK) * 128
_R = 7813 * 128
_TAIL = _TILES * 128


def _detile_body(a3, b3, t3a, t3b, o3a, o3b,
                 bufa, bufb, ra0, ra1, ra2, rb0, rb1, rb2, tb3,
                 rsem, wsem):
    wid = lax.axis_index("s") * _NC + lax.axis_index("c")
    base = wid * _TPW * 128
    bufs = (bufa, bufb)
    rowsets = ((ra0, ra1, ra2), (rb0, rb1, rb2))

    units = [(k, src, out)
             for k in range(_ACHUNK)
             for src, out in ((a3, o3a), (b3, o3b))]

    def read(i):
        k, src, _ = units[i]
        return pltpu.async_copy(
            src.at[:, pl.ds(base + k * _CW, _CW)], bufs[i % 2], rsem)

    def extract(buf, rows):
        def step(i, _):
            for u in range(4):
                o = (i * 4 + u) * _L
                for c in range(3):
                    rows[c][pl.ds(o, _L)] = buf[c, pl.ds(o, _L)]
            return 0
        lax.fori_loop(0, _CW // (_L * 4), step, 0)

    pending_read = read(0)
    pending_writes = ([], [])
    for i, (k, src, out) in enumerate(units):
        pending_read.wait()
        if i + 1 < len(units):
            pending_read = read(i + 1)
        for w in pending_writes[i % 2]:
            w.wait()
        rows = rowsets[i % 2]
        extract(bufs[i % 2], rows)
        pending_writes = (
            [pltpu.async_copy(
                rows[c], out.at[pl.ds(c * _R + base + k * _CW, _CW)], wsem)
             for c in range(3)] if i % 2 == 0 else pending_writes[0],
            [pltpu.async_copy(
                rows[c], out.at[pl.ds(c * _R + base + k * _CW, _CW)], wsem)
             for c in range(3)] if i % 2 == 1 else pending_writes[1],
        )
    for ws in pending_writes:
        for w in ws:
            w.wait()

    @pl.when(wid < 4)
    def _():
        toff = (_TILES - 4) * 128 + wid * 128
        rows = rowsets[0]
        for src, out in ((a3, o3a), (b3, o3b)):
            pltpu.sync_copy(src.at[:, pl.ds(toff, 128)],
                            bufa.at[:, pl.ds(0, 128)])

            def tstep(i, _):
                for c in range(3):
                    rows[c][pl.ds(i * _L, _L)] = bufa[c, pl.ds(i * _L, _L)]
                return 0
            lax.fori_loop(0, 128 // _L, tstep, 0)
            for c in range(3):
                pltpu.sync_copy(rows[c].at[pl.ds(0, 128)],
                                out.at[pl.ds(c * _R + toff, 128)])

    @pl.when(wid == 4)
    def _():
        pltpu.sync_copy(t3a, tb3)
        for c in range(3):
            pltpu.sync_copy(tb3.at[pl.ds(c * 128, 128)],
                            o3a.at[pl.ds(c * _R + _TAIL, 128)])

    @pl.when(wid == 5)
    def _():
        pltpu.sync_copy(t3b, tb3)
        for c in range(3):
            pltpu.sync_copy(tb3.at[pl.ds(c * 128, 128)],
                            o3b.at[pl.ds(c * _R + _TAIL, 128)])


def _gather_body(idx_hbm, i4, f3a, f3b, *refs):
    outs = refs[:_NCOL]
    idx_v, cols_v, sem = refs[_NCOL:]
    wid = lax.axis_index("s") * _NC + lax.axis_index("c")
    base = wid * _BPW

    pltpu.sync_copy(idx_hbm.at[pl.ds(wid * _NCHUNK, _NCHUNK)], idx_v)

    srcs = ([i4.at[c] for c in range(4)]
            + [f3a.at[pl.ds(c * _R, _R)] for c in range(3)]
            + [f3b.at[pl.ds(c * _R, _R)] for c in range(3)])
    copies = []
    for c in range(_NCOL):
        for k in range(_NCHUNK):
            copies.append(pltpu.async_copy(
                srcs[c].at[idx_v.at[k]],
                cols_v.at[c].at[pl.ds(k * _CHUNK, _CHUNK)],
                sem))
    for cp in copies:
        cp.wait()

    writes = [pltpu.async_copy(cols_v.at[c], outs[c].at[pl.ds(base, _BPW)],
                               sem)
              for c in range(_NCOL)]
    for w in writes:
        w.wait()


@jax.jit
def _run(camera_idx, intrinsic_deltas, rotation_deltas, translation_deltas):
    mesh = plsc.VectorSubcoreMesh(core_axis_name="c", subcore_axis_name="s")

    tail3a = jnp.ravel(
        jnp.pad(rotation_deltas[_TAIL:], ((0, 64), (0, 0))).T)
    tail3b = jnp.ravel(
        jnp.pad(translation_deltas[_TAIL:], ((0, 64), (0, 0))).T)

    detile = functools.partial(
        pl.kernel,
        out_type=(jax.ShapeDtypeStruct((3 * _R,), jnp.float32),
                  jax.ShapeDtypeStruct((3 * _R,), jnp.float32)),
        mesh=mesh,
        scratch_types=[
            pltpu.VMEM((3, _CW), jnp.float32),
            pltpu.VMEM((3, _CW), jnp.float32),
            pltpu.VMEM((_CW,), jnp.float32),
            pltpu.VMEM((_CW,), jnp.float32),
            pltpu.VMEM((_CW,), jnp.float32),
            pltpu.VMEM((_CW,), jnp.float32),
            pltpu.VMEM((_CW,), jnp.float32),
            pltpu.VMEM((_CW,), jnp.float32),
            pltpu.VMEM((3 * 128,), jnp.float32),
            pltpu.SemaphoreType.DMA,
            pltpu.SemaphoreType.DMA,
        ],
        compiler_params=pltpu.CompilerParams(
            use_tc_tiling_on_sc=True, needs_layout_passes=False),
    )(_detile_body)
    f3a, f3b = detile(rotation_deltas.T, translation_deltas.T,
                      tail3a, tail3b)

    gather = functools.partial(
        pl.kernel,
        out_type=tuple(jax.ShapeDtypeStruct((_BATCH,), jnp.float32)
                       for _ in range(_NCOL)),
        mesh=mesh,
        scratch_types=[
            pltpu.VMEM((_NCHUNK, _CHUNK), jnp.int32),
            pltpu.VMEM((_NCOL, _BPW), jnp.float32),
            pltpu.SemaphoreType.DMA,
        ],
        compiler_params=pltpu.CompilerParams(
            use_tc_tiling_on_sc=False, needs_layout_passes=False),
    )(_gather_body)
    idx2 = camera_idx.reshape(_NW * _NCHUNK, _CHUNK)
    cols = gather(idx2, intrinsic_deltas.T, f3a, f3b)
    return jnp.stack(cols, axis=1)


def kernel(camera_idx, intrinsic_deltas, rotation_deltas, translation_deltas):
    return _run(camera_idx.astype(jnp.int32), intrinsic_deltas,
                rotation_deltas, translation_deltas)
